# baseline (speedup 1.0000x reference)
import jax
import jax.numpy as jnp
from jax import lax
from jax.experimental import pallas as pl
from jax.experimental.pallas import tpu as pltpu
from jax.experimental.pallas import tpu_sc as plsc

_NC, _NS, _L = 2, 16, 16
_NW = _NC * _NS

_CH = 16
_TC_SEQ_BLOCK = 2048


def _sc_add(emb_flat, pos_flat, batch, seq, hid):
    rows_per_w = seq // _NW
    nchunk = rows_per_w // _CH
    chw = _CH * hid

    def body(emb_hbm, pos_hbm, out_hbm, pos_v, buf_v):
        wid = lax.axis_index("s") * _NC + lax.axis_index("c")
        base = wid * rows_per_w * hid

        @pl.loop(0, nchunk)
        def _chunk_loop(ci):
            off = base + ci * chw
            pltpu.sync_copy(pos_hbm.at[pl.ds(off, chw)], pos_v)
            for b in range(batch):
                eoff = b * (seq * hid) + off
                pltpu.sync_copy(emb_hbm.at[pl.ds(eoff, chw)], buf_v)

                @plsc.parallel_loop(0, chw // _L, unroll=8)
                def _vec_loop(k):
                    sl = pl.ds(k * _L, _L)
                    buf_v[sl] = buf_v[sl] + pos_v[sl]

                pltpu.sync_copy(buf_v, out_hbm.at[pl.ds(eoff, chw)])

    fn = pl.kernel(
        body,
        out_type=jax.ShapeDtypeStruct((batch * seq * hid,), jnp.float32),
        mesh=plsc.VectorSubcoreMesh(core_axis_name="c", subcore_axis_name="s"),
        scratch_types=[
            pltpu.VMEM((chw,), jnp.float32),
            pltpu.VMEM((chw,), jnp.float32),
        ],
    )
    return fn(emb_flat, pos_flat)


def _tc_add_kernel(emb_ref, pos_ref, out_ref):
    out_ref[...] = emb_ref[...] + pos_ref[...]


def _tc_add(embeddings, pos_table):
    batch, seq, hid = embeddings.shape
    grid = (seq // _TC_SEQ_BLOCK, batch)
    return pl.pallas_call(
        _tc_add_kernel,
        grid=grid,
        in_specs=[
            pl.BlockSpec((1, _TC_SEQ_BLOCK, hid), lambda i, j: (j, i, 0)),
            pl.BlockSpec((_TC_SEQ_BLOCK, hid), lambda i, j: (i, 0)),
        ],
        out_specs=pl.BlockSpec((1, _TC_SEQ_BLOCK, hid), lambda i, j: (j, i, 0)),
        out_shape=jax.ShapeDtypeStruct((batch, seq, hid), embeddings.dtype),
        compiler_params=pltpu.CompilerParams(
            dimension_semantics=("arbitrary", "arbitrary"),
        ),
    )(embeddings, pos_table)


def kernel(embeddings, pos_table):
    batch, seq, hid = embeddings.shape
    out_flat = _sc_add(
        embeddings.reshape(-1), pos_table.reshape(-1), batch, seq, hid
    )
    return out_flat.reshape(batch, seq, hid)


# speedup vs baseline: 1.2263x; 1.2263x over previous
import jax
import jax.numpy as jnp
from jax import lax
from jax.experimental import pallas as pl
from jax.experimental.pallas import tpu as pltpu
from jax.experimental.pallas import tpu_sc as plsc

_NC, _NS, _L = 2, 16, 16
_NW = _NC * _NS

_CH = 16
_TC_SEQ_BLOCK = 2048


def _sc_add(emb_flat, pos_flat, batch, seq, hid):
    rows_per_w = seq // _NW
    nchunk = rows_per_w // _CH
    chw = _CH * hid

    def body(emb_hbm, pos_hbm, out_hbm,
             b0, b1, b2, b3, pos_v,
             is0, is1, is2, is3, os0, os1, os2, os3, ps):
        bufs = (b0, b1, b2, b3)
        in_sems = (is0, is1, is2, is3)
        out_sems = (os0, os1, os2, os3)
        wid = lax.axis_index("s") * _NC + lax.axis_index("c")
        base = wid * rows_per_w * hid

        def compute_add(buf):
            @plsc.parallel_loop(0, chw // _L, unroll=8)
            def _vec_loop(k):
                sl = pl.ds(k * _L, _L)
                buf[sl] = buf[sl] + pos_v[sl]

        @pl.loop(0, nchunk)
        def _chunk_loop(ci):
            off = base + ci * chw
            pltpu.async_copy(pos_hbm.at[pl.ds(off, chw)], pos_v, ps)
            for b in range(batch):
                eoff = b * (seq * hid) + off

                @pl.when(ci > 0)
                def _drain_prev_out():
                    peoff = b * (seq * hid) + off - chw
                    pltpu.make_async_copy(
                        bufs[b], out_hbm.at[pl.ds(peoff, chw)], out_sems[b]
                    ).wait()

                pltpu.async_copy(
                    emb_hbm.at[pl.ds(eoff, chw)], bufs[b], in_sems[b]
                )
            pltpu.make_async_copy(
                pos_hbm.at[pl.ds(off, chw)], pos_v, ps
            ).wait()
            for b in range(batch):
                eoff = b * (seq * hid) + off
                pltpu.make_async_copy(
                    emb_hbm.at[pl.ds(eoff, chw)], bufs[b], in_sems[b]
                ).wait()
                compute_add(bufs[b])
                pltpu.async_copy(
                    bufs[b], out_hbm.at[pl.ds(eoff, chw)], out_sems[b]
                )

        last_off = base + (nchunk - 1) * chw
        for b in range(batch):
            leoff = b * (seq * hid) + last_off
            pltpu.make_async_copy(
                bufs[b], out_hbm.at[pl.ds(leoff, chw)], out_sems[b]
            ).wait()

    dma = pltpu.SemaphoreType.DMA
    fn = pl.kernel(
        body,
        out_type=jax.ShapeDtypeStruct((batch * seq * hid,), jnp.float32),
        mesh=plsc.VectorSubcoreMesh(core_axis_name="c", subcore_axis_name="s"),
        scratch_types=(
            [pltpu.VMEM((chw,), jnp.float32)] * (batch + 1)
            + [dma] * (2 * batch + 1)
        ),
    )
    return fn(emb_flat, pos_flat)


def _tc_add_kernel(emb_ref, pos_ref, out_ref):
    out_ref[...] = emb_ref[...] + pos_ref[...]


def _tc_add(embeddings, pos_table):
    batch, seq, hid = embeddings.shape
    grid = (seq // _TC_SEQ_BLOCK, batch)
    return pl.pallas_call(
        _tc_add_kernel,
        grid=grid,
        in_specs=[
            pl.BlockSpec((1, _TC_SEQ_BLOCK, hid), lambda i, j: (j, i, 0)),
            pl.BlockSpec((_TC_SEQ_BLOCK, hid), lambda i, j: (i, 0)),
        ],
        out_specs=pl.BlockSpec((1, _TC_SEQ_BLOCK, hid), lambda i, j: (j, i, 0)),
        out_shape=jax.ShapeDtypeStruct((batch, seq, hid), embeddings.dtype),
        compiler_params=pltpu.CompilerParams(
            dimension_semantics=("arbitrary", "arbitrary"),
        ),
    )(embeddings, pos_table)


def kernel(embeddings, pos_table):
    batch, seq, hid = embeddings.shape
    out_flat = _sc_add(
        embeddings.reshape(-1), pos_table.reshape(-1), batch, seq, hid
    )
    return out_flat.reshape(batch, seq, hid)


# speedup vs baseline: 5.2719x; 4.2991x over previous
import jax
import jax.numpy as jnp
from jax import lax
from jax.experimental import pallas as pl
from jax.experimental.pallas import tpu as pltpu
from jax.experimental.pallas import tpu_sc as plsc

_NC, _NS, _L = 2, 16, 16
_NW = _NC * _NS

_CH = 16
_TC_SEQ_BLOCK = 1024


def _sc_add(emb_flat, pos_flat, batch, seq, hid):
    rows_per_w = seq // _NW
    nchunk = rows_per_w // _CH
    chw = _CH * hid

    def body(emb_hbm, pos_hbm, out_hbm,
             b0, b1, b2, b3, pos_v,
             is0, is1, is2, is3, os0, os1, os2, os3, ps):
        bufs = (b0, b1, b2, b3)
        in_sems = (is0, is1, is2, is3)
        out_sems = (os0, os1, os2, os3)
        wid = lax.axis_index("s") * _NC + lax.axis_index("c")
        base = wid * rows_per_w * hid

        def compute_add(buf):
            @plsc.parallel_loop(0, chw // _L, unroll=8)
            def _vec_loop(k):
                sl = pl.ds(k * _L, _L)
                buf[sl] = buf[sl] + pos_v[sl]

        @pl.loop(0, nchunk)
        def _chunk_loop(ci):
            off = base + ci * chw
            pltpu.async_copy(pos_hbm.at[pl.ds(off, chw)], pos_v, ps)
            for b in range(batch):
                eoff = b * (seq * hid) + off

                @pl.when(ci > 0)
                def _drain_prev_out():
                    peoff = b * (seq * hid) + off - chw
                    pltpu.make_async_copy(
                        bufs[b], out_hbm.at[pl.ds(peoff, chw)], out_sems[b]
                    ).wait()

                pltpu.async_copy(
                    emb_hbm.at[pl.ds(eoff, chw)], bufs[b], in_sems[b]
                )
            pltpu.make_async_copy(
                pos_hbm.at[pl.ds(off, chw)], pos_v, ps
            ).wait()
            for b in range(batch):
                eoff = b * (seq * hid) + off
                pltpu.make_async_copy(
                    emb_hbm.at[pl.ds(eoff, chw)], bufs[b], in_sems[b]
                ).wait()
                compute_add(bufs[b])
                pltpu.async_copy(
                    bufs[b], out_hbm.at[pl.ds(eoff, chw)], out_sems[b]
                )

        last_off = base + (nchunk - 1) * chw
        for b in range(batch):
            leoff = b * (seq * hid) + last_off
            pltpu.make_async_copy(
                bufs[b], out_hbm.at[pl.ds(leoff, chw)], out_sems[b]
            ).wait()

    dma = pltpu.SemaphoreType.DMA
    fn = pl.kernel(
        body,
        out_type=jax.ShapeDtypeStruct((batch * seq * hid,), jnp.float32),
        mesh=plsc.VectorSubcoreMesh(core_axis_name="c", subcore_axis_name="s"),
        scratch_types=(
            [pltpu.VMEM((chw,), jnp.float32)] * (batch + 1)
            + [dma] * (2 * batch + 1)
        ),
    )
    return fn(emb_flat, pos_flat)


def _tc_add_kernel(emb_ref, pos_ref, out_ref):
    out_ref[...] = emb_ref[...] + pos_ref[...]


def _tc_add(embeddings, pos_table):
    batch, seq, hid = embeddings.shape
    grid = (seq // _TC_SEQ_BLOCK, batch)
    return pl.pallas_call(
        _tc_add_kernel,
        grid=grid,
        in_specs=[
            pl.BlockSpec((1, _TC_SEQ_BLOCK, hid), lambda i, j: (j, i, 0)),
            pl.BlockSpec((_TC_SEQ_BLOCK, hid), lambda i, j: (i, 0)),
        ],
        out_specs=pl.BlockSpec((1, _TC_SEQ_BLOCK, hid), lambda i, j: (j, i, 0)),
        out_shape=jax.ShapeDtypeStruct((batch, seq, hid), embeddings.dtype),
        compiler_params=pltpu.CompilerParams(
            dimension_semantics=("arbitrary", "arbitrary"),
        ),
    )(embeddings, pos_table)


def kernel(embeddings, pos_table):
    return _tc_add(embeddings, pos_table)


# speedup vs baseline: 5.4929x; 1.0419x over previous
import jax
import jax.numpy as jnp
from jax.experimental import pallas as pl
from jax.experimental.pallas import tpu as pltpu

_SEQ_BLOCK = 2048


def _add_kernel(emb_ref, pos_ref, out_ref):
    out_ref[...] = emb_ref[...] + pos_ref[...]


def kernel(embeddings, pos_table):
    batch, seq, hid = embeddings.shape
    grid = (seq // _SEQ_BLOCK, batch)
    return pl.pallas_call(
        _add_kernel,
        grid=grid,
        in_specs=[
            pl.BlockSpec((1, _SEQ_BLOCK, hid), lambda i, j: (j, i, 0)),
            pl.BlockSpec((_SEQ_BLOCK, hid), lambda i, j: (i, 0)),
        ],
        out_specs=pl.BlockSpec((1, _SEQ_BLOCK, hid), lambda i, j: (j, i, 0)),
        out_shape=jax.ShapeDtypeStruct((batch, seq, hid), embeddings.dtype),
        compiler_params=pltpu.CompilerParams(
            dimension_semantics=("arbitrary", "arbitrary"),
        ),
    )(embeddings, pos_table)
